# baseline (speedup 1.0000x reference)
import functools

import jax
import jax.numpy as jnp
from jax import lax
from jax.experimental import pallas as pl
from jax.experimental.pallas import tpu as pltpu
from jax.experimental.pallas import tpu_sc as plsc

NUM_FIELDS = 26
VOCAB = 100000
EMBED_DIM = 32
BATCH = 16384

_INFO = plsc.get_sparse_core_info()
_NC, _NS = _INFO.num_cores, _INFO.num_subcores
_NW = _NC * _NS
_BPW = BATCH // _NW
_IW = 128
_GPF = _BPW // _IW


def _make_gather():
    mesh = plsc.VectorSubcoreMesh(core_axis_name="c", subcore_axis_name="s")

    @functools.partial(
        pl.kernel,
        mesh=mesh,
        out_type=jax.ShapeDtypeStruct((BATCH, NUM_FIELDS, EMBED_DIM),
                                      jnp.float32),
        scratch_types=[
            pltpu.VMEM((NUM_FIELDS, _BPW), jnp.int32),
            pltpu.VMEM((_BPW, EMBED_DIM), jnp.float32),
            pltpu.VMEM((_BPW, EMBED_DIM), jnp.float32),
            pltpu.SemaphoreType.DMA,
            pltpu.SemaphoreType.DMA,
            pltpu.SemaphoreType.DMA,
            pltpu.SemaphoreType.DMA,
        ],
        compiler_params=pltpu.CompilerParams(use_tc_tiling_on_sc=False),
    )
    def gather_kernel(tab_hbm, idx_hbm, out_hbm, idx_v, rows0, rows1,
                      sg0, sg1, sw0, sw1):
        wid = lax.axis_index("s") * _NC + lax.axis_index("c")
        base = wid * _BPW
        pltpu.sync_copy(idx_hbm.at[:, pl.ds(base, _BPW)], idx_v)

        def fire(f, buf, sem):
            for j in range(_GPF):
                pltpu.async_copy(
                    tab_hbm.at[f].at[idx_v.at[f, pl.ds(j * _IW, _IW)]],
                    buf.at[pl.ds(j * _IW, _IW)],
                    sem,
                )

        def drain(buf, sem):
            pltpu.make_async_copy(out_hbm.at[pl.ds(0, _BPW), 0], buf, sem).wait()

        def write(f, buf, sem):
            pltpu.async_copy(buf, out_hbm.at[pl.ds(base, _BPW), f], sem)

        fire(0, rows0, sg0)

        def pair(k, _):
            f0 = 2 * k

            @pl.when(k >= 1)
            def _():
                drain(rows1, sw1)
            fire(f0 + 1, rows1, sg1)
            drain(rows0, sg0)
            write(f0, rows0, sw0)

            @pl.when(f0 + 2 < NUM_FIELDS)
            def _():
                drain(rows0, sw0)
                fire(f0 + 2, rows0, sg0)
            drain(rows1, sg1)
            write(f0 + 1, rows1, sw1)
            return ()

        lax.fori_loop(0, NUM_FIELDS // 2, pair, (), unroll=False)
        drain(rows0, sw0)
        drain(rows1, sw1)

    return gather_kernel


_gather = _make_gather()


def kernel(x, tables):
    idx_t = x.astype(jnp.int32).T
    out = _gather(tables, idx_t)
    return out.reshape(BATCH, NUM_FIELDS * EMBED_DIM)
